# baseline (speedup 1.0000x reference)
import functools

import jax
import jax.numpy as jnp
from jax import lax
from jax.experimental import pallas as pl
from jax.experimental.pallas import tpu as pltpu
from jax.experimental.pallas import tpu_sc as plsc

N = 10000
E = 160000
G = 64
NB = 2
NS = 16
KB = 64
NBATCH = 160
NBUF = 4
EP = NS * NBATCH * KB
N2 = 10112
RPS = N2 // NS
BI = 1000
BJ = 512
RSEG = 80
NSEG_CH = N // RSEG

_f32 = jnp.float32
_mesh = plsc.VectorSubcoreMesh(core_axis_name="c", subcore_axis_name="s")


def _deg_body(dstr, zeros_hbm, ones_hbm, out, idx_d, ones_v, acc):
    c = lax.axis_index("c")
    s = lax.axis_index("s")
    pltpu.sync_copy(dstr.at[c, s], idx_d)
    pltpu.sync_copy(ones_hbm, ones_v)
    pltpu.sync_copy(zeros_hbm.at[pl.ds(s * RPS, RPS)],
                    acc.at[pl.ds(s * RPS, RPS)])
    plsc.subcore_barrier()

    def body(b, carry):
        pltpu.sync_copy(ones_v, acc.at[idx_d.at[b]], add=True)
        return carry

    lax.fori_loop(0, NBATCH, body, 0)
    plsc.subcore_barrier()
    pltpu.sync_copy(acc.at[pl.ds(s * RPS, RPS)],
                    out.at[c, pl.ds(s * RPS, RPS)])


def _sc_degree(dstr, zeros_init, ones_blk):
    return pl.kernel(
        _deg_body,
        out_type=jax.ShapeDtypeStruct((NB, N2, 128), _f32),
        mesh=_mesh,
        scratch_types=[
            pltpu.VMEM((NBATCH, KB), jnp.int32),
            pltpu.VMEM((KB, 128), _f32),
            pltpu.VMEM_SHARED((N2, 128), _f32),
        ],
    )(dstr, zeros_init, ones_blk)


NRES = 4
HB = NBATCH // NRES


def _agg_body(nchunk, y3, srcr, dstr, out, idx_s, idx_d, rows, acc, gsems,
              ssems):
    c = lax.axis_index("c")
    s = lax.axis_index("s")
    for ch in range(nchunk):
        pltpu.sync_copy(y3.at[c, ch, pl.ds(s * RPS, RPS)],
                        acc.at[pl.ds(s * RPS, RPS)])
        plsc.subcore_barrier()

        tbl = y3.at[c, ch]
        for half in range(NRES):
            pltpu.sync_copy(srcr.at[c, s, pl.ds(half * HB, HB)], idx_s)
            pltpu.sync_copy(dstr.at[c, s, pl.ds(half * HB, HB)], idx_d)
            for j in range(2):
                pltpu.async_copy(tbl.at[idx_s.at[j]], rows.at[j],
                                 gsems.at[j])

            def body(t, carry):
                for j4 in range(NBUF):
                    b = t * NBUF + j4
                    jf = (j4 + 2) % NBUF

                    @pl.when(b >= 2)
                    def _():
                        pltpu.make_async_copy(
                            rows.at[jf], acc.at[idx_d.at[b - 2]],
                            ssems.at[jf]).wait()

                    @pl.when(b + 2 < HB)
                    def _():
                        pltpu.async_copy(tbl.at[idx_s.at[b + 2]],
                                         rows.at[jf], gsems.at[jf])

                    pltpu.make_async_copy(tbl.at[idx_s.at[b]], rows.at[j4],
                                          gsems.at[j4]).wait()
                    pltpu.async_copy(rows.at[j4], acc.at[idx_d.at[b]],
                                     ssems.at[j4], add=True)
                return carry

            lax.fori_loop(0, HB // NBUF, body, 0)
            for j in range(2):
                b = HB - 2 + j
                pltpu.make_async_copy(rows.at[b % NBUF],
                                      acc.at[idx_d.at[b]],
                                      ssems.at[b % NBUF]).wait()
        plsc.subcore_barrier()
        pltpu.sync_copy(acc.at[pl.ds(s * RPS, RPS)],
                        out.at[c, ch, pl.ds(s * RPS, RPS)])
        plsc.subcore_barrier()


def _sc_agg(y3, srcr, dstr):
    nchunk = y3.shape[1]
    return pl.kernel(
        functools.partial(_agg_body, nchunk),
        out_type=jax.ShapeDtypeStruct((NB, nchunk, N2, 128), _f32),
        mesh=_mesh,
        scratch_types=[
            pltpu.VMEM((HB, KB), jnp.int32),
            pltpu.VMEM((HB, KB), jnp.int32),
            pltpu.VMEM((NBUF, KB, 128), _f32),
            pltpu.VMEM_SHARED((N2, 128), _f32),
            pltpu.SemaphoreType.DMA((NBUF,)),
            pltpu.SemaphoreType.DMA((NBUF,)),
        ],
    )(y3, srcr, dstr)


def _segmax_body(z2, batch, neg_hbm, out, buf, bsm, acc):
    c = lax.axis_index("c")
    s = lax.axis_index("s")
    pltpu.sync_copy(neg_hbm, acc)
    pltpu.sync_copy(batch.at[c, 0], bsm)

    def chunk_body(r, carry):
        pltpu.sync_copy(z2.at[c, pl.ds(r * RSEG, RSEG), pl.ds(s * 128, 128)],
                        buf)

        def grp_body(gi, carry2):
            gvec = bsm[pl.ds(r * RSEG + gi * 16, 16)]
            for jj in range(16):
                g = gvec[jj]
                for v in range(8):
                    sl = pl.ds(v * 16, 16)
                    acc[g, sl] = jnp.maximum(acc[g, sl], buf[gi * 16 + jj, sl])
            return carry2

        lax.fori_loop(0, RSEG // 16, grp_body, 0)
        return carry

    lax.fori_loop(0, NSEG_CH, chunk_body, 0)
    pltpu.sync_copy(acc, out.at[c, :, pl.ds(s * 128, 128)])


def _sc_segmax(z2, batch, neg):
    return pl.kernel(
        _segmax_body,
        out_type=jax.ShapeDtypeStruct((NB, G, 2048), _f32),
        mesh=_mesh,
        scratch_types=[
            pltpu.VMEM((RSEG, 128), _f32),
            pltpu.VMEM((N,), jnp.int32),
            pltpu.VMEM((G, 128), _f32),
        ],
    )(z2, batch, neg)


def _prescale_body(x_ref, deg_ref, dinv_ref, y1_ref):
    d = deg_ref[0, :, 0:1] + 1.0
    dv = lax.rsqrt(d)
    dinv_ref[0] = dv
    xv = x_ref[0] * dv
    for ci in range(4):
        y1_ref[0, ci] = xv[:, 128 * ci:128 * (ci + 1)]


def _tc_prescale(x2, deg2):
    nI = N // BI
    return pl.pallas_call(
        _prescale_body,
        grid=(NB, nI),
        in_specs=[
            pl.BlockSpec((1, BI, 512), lambda b, i: (b, i, 0)),
            pl.BlockSpec((1, BI, 128), lambda b, i: (b, i, 0)),
        ],
        out_specs=[
            pl.BlockSpec((1, BI, 1), lambda b, i: (b, i, 0)),
            pl.BlockSpec((1, 4, BI, 128), lambda b, i: (b, 0, i, 0)),
        ],
        out_shape=[
            jax.ShapeDtypeStruct((NB, N, 1), _f32),
            jax.ShapeDtypeStruct((NB, 4, N2, 128), _f32),
        ],
    )(x2, deg2)


def _mm_body(a_ref, w_ref, b_ref, dinv_ref, out_ref, *,
             cin, relu, post, chunked):
    acc = jnp.dot(a_ref[0, 0], w_ref[0][:128],
                  preferred_element_type=_f32)
    for ci in range(1, cin):
        acc += jnp.dot(a_ref[0, ci], w_ref[0][128 * ci:128 * (ci + 1)],
                       preferred_element_type=_f32)
    dv = dinv_ref[0]
    t = acc * dv + b_ref[0]
    if relu:
        t = jnp.maximum(t, 0.0)
    if post:
        t = t * dv
    if chunked:
        for q in range(BJ // 128):
            out_ref[0, q] = t[:, 128 * q:128 * (q + 1)]
    else:
        out_ref[0] = t


def _tc_gcn_matmul(a3, w2, b2, dinv2, relu, post, chunked):
    cin = a3.shape[1]
    wout = w2.shape[2]
    nI, nJ = N // BI, wout // BJ
    body = functools.partial(_mm_body, cin=cin, relu=relu, post=post,
                             chunked=chunked)
    nq = BJ // 128
    if chunked:
        out_spec = pl.BlockSpec((1, nq, BI, 128), lambda b, i, j: (b, j, i, 0))
        out_shape = jax.ShapeDtypeStruct((NB, wout // 128, N2, 128), _f32)
    else:
        out_spec = pl.BlockSpec((1, BI, BJ), lambda b, i, j: (b, i, j))
        out_shape = jax.ShapeDtypeStruct((NB, N, wout), _f32)
    return pl.pallas_call(
        body,
        grid=(NB, nI, nJ),
        in_specs=[
            pl.BlockSpec((1, cin, BI, 128), lambda b, i, j: (b, 0, i, 0)),
            pl.BlockSpec((1, 128 * cin, BJ), lambda b, i, j: (b, 0, j)),
            pl.BlockSpec((1, 1, BJ), lambda b, i, j: (b, 0, j)),
            pl.BlockSpec((1, BI, 1), lambda b, i, j: (b, i, 0)),
        ],
        out_specs=out_spec,
        out_shape=out_shape,
        compiler_params=pltpu.CompilerParams(
            dimension_semantics=("parallel", "parallel", "parallel")),
    )(a3, w2, b2, dinv2)


def _head_body(p_ref, w1_ref, b1_ref, w2_ref, b2_ref, o1_ref, o2_ref):
    q = p_ref[0] + p_ref[1]
    z = jnp.maximum(jnp.dot(q, w1_ref[...], preferred_element_type=_f32)
                    + b1_ref[...], 0.0)
    o = jnp.dot(z, w2_ref[...], preferred_element_type=_f32) + b2_ref[...]
    o1_ref[...] = o
    o2_ref[...] = jax.nn.sigmoid(o)


def _tc_head(p2, w1, b1, w2p, b2p):
    nout = w2p.shape[1]
    return pl.pallas_call(
        _head_body,
        out_shape=[
            jax.ShapeDtypeStruct((G, nout), _f32),
            jax.ShapeDtypeStruct((G, nout), _f32),
        ],
    )(p2, w1, b1, w2p, b2p)


def _prep_edges(ei):
    order = jnp.argsort(ei[1])
    pad = EP - E
    src = jnp.concatenate([ei[0][order], jnp.zeros((pad,), jnp.int32)])
    dst = jnp.concatenate([ei[1][order], jnp.full((pad,), N + 8, jnp.int32)])
    return src.reshape(NS, NBATCH, KB), dst.reshape(NS, NBATCH, KB)


def kernel(x_s, x_t, edge_index_s, edge_index_t, xs_batch, xt_batch,
           W_enc1, b_enc1, W_enc2, b_enc2,
           W_r1a, b_r1a, W_r1b, b_r1b,
           W_r2a, b_r2a, W_r2b, b_r2b,
           W_l1, b_l1, W_l2, b_l2):
    x2 = jnp.stack([x_s, x_t])
    src_s, dst_s = _prep_edges(edge_index_s)
    src_t, dst_t = _prep_edges(edge_index_t)
    src2 = jnp.stack([src_s, src_t])
    dst2 = jnp.stack([dst_s, dst_t])
    batch2 = jnp.stack([xs_batch, xt_batch])[:, None, :]

    w_enc = jnp.stack([W_enc1, W_enc2])
    b_enc = jnp.stack([b_enc1, b_enc2])[:, None, :]
    w_ra = jnp.stack([W_r1a, W_r2a])
    b_ra = jnp.stack([b_r1a, b_r2a])[:, None, :]
    w_rb = jnp.stack([W_r1b, W_r2b])
    b_rb = jnp.stack([b_r1b, b_r2b])[:, None, :]

    b_l2r = b_l2[None, :]
    b_l1r = b_l1[None, :]

    zeros_init = jnp.zeros((N2, 128), _f32)
    ones_blk = jnp.ones((KB, 128), _f32)
    neg = jnp.full((G, 128), -jnp.inf, _f32)

    deg2 = _sc_degree(dst2, zeros_init, ones_blk)
    dinv2, y1 = _tc_prescale(x2, deg2)
    a1 = _sc_agg(y1, src2, dst2)
    y2 = _tc_gcn_matmul(a1, w_enc, b_enc, dinv2, relu=False, post=True,
                        chunked=True)
    a2 = _sc_agg(y2, src2, dst2)
    y3 = _tc_gcn_matmul(a2, w_ra, b_ra, dinv2, relu=True, post=True,
                        chunked=True)
    a3 = _sc_agg(y3, src2, dst2)
    z2 = _tc_gcn_matmul(a3, w_rb, b_rb, dinv2, relu=True, post=False,
                        chunked=False)
    p2 = _sc_segmax(z2, batch2, neg)
    z, sig = _tc_head(p2, W_l1, b_l1r, W_l2, b_l2r)
    return (z, sig)


# speedup vs baseline: 1.1131x; 1.1131x over previous
import functools

import jax
import jax.numpy as jnp
from jax import lax
from jax.experimental import pallas as pl
from jax.experimental.pallas import tpu as pltpu
from jax.experimental.pallas import tpu_sc as plsc

N = 10000
E = 160000
G = 64
NB = 2
NS = 16
KB = 64
NBATCH = 160
NBUF = 4
EP = NS * NBATCH * KB
N2 = 10112
RPS = N2 // NS
BI = 1000
BJ = 512
RSEG = 80
NSEG_CH = N // RSEG

_f32 = jnp.float32
_mesh = plsc.VectorSubcoreMesh(core_axis_name="c", subcore_axis_name="s")


def _deg_body(dstr, zeros_hbm, ones_hbm, out, idx_d, ones_v, acc):
    c = lax.axis_index("c")
    s = lax.axis_index("s")
    pltpu.sync_copy(dstr.at[c, s], idx_d)
    pltpu.sync_copy(ones_hbm, ones_v)
    pltpu.sync_copy(zeros_hbm.at[pl.ds(s * RPS, RPS)],
                    acc.at[pl.ds(s * RPS, RPS)])
    plsc.subcore_barrier()

    def body(b, carry):
        pltpu.sync_copy(ones_v, acc.at[idx_d.at[b]], add=True)
        return carry

    lax.fori_loop(0, NBATCH, body, 0)
    plsc.subcore_barrier()
    pltpu.sync_copy(acc.at[pl.ds(s * RPS, RPS)],
                    out.at[c, pl.ds(s * RPS, RPS)])


def _sc_degree(dstr, zeros_init, ones_blk):
    return pl.kernel(
        _deg_body,
        out_type=jax.ShapeDtypeStruct((NB, N2, 128), _f32),
        mesh=_mesh,
        scratch_types=[
            pltpu.VMEM((NBATCH, KB), jnp.int32),
            pltpu.VMEM((KB, 128), _f32),
            pltpu.VMEM_SHARED((N2, 128), _f32),
        ],
    )(dstr, zeros_init, ones_blk)


NRES = 4
HB = NBATCH // NRES


def _agg_body(nchunk, y3, srcr, dstr, out, idx_s, idx_d, rows, acc, gsems,
              ssems):
    c = lax.axis_index("c")
    s = lax.axis_index("s")
    for ch in range(nchunk):
        pltpu.sync_copy(y3.at[c, ch, pl.ds(s * RPS, RPS)],
                        acc.at[pl.ds(s * RPS, RPS)])
        plsc.subcore_barrier()

        tbl = y3.at[c, ch]
        for half in range(NRES):
            pltpu.sync_copy(srcr.at[c, s, pl.ds(half * HB, HB)], idx_s)
            pltpu.sync_copy(dstr.at[c, s, pl.ds(half * HB, HB)], idx_d)
            for j in range(2):
                pltpu.async_copy(tbl.at[idx_s.at[j]], rows.at[j],
                                 gsems.at[j])

            def body(t, carry):
                for j4 in range(NBUF):
                    b = t * NBUF + j4
                    jf = (j4 + 2) % NBUF

                    @pl.when(b >= 2)
                    def _():
                        pltpu.make_async_copy(
                            rows.at[jf], acc.at[idx_d.at[b - 2]],
                            ssems.at[jf]).wait()

                    @pl.when(b + 2 < HB)
                    def _():
                        pltpu.async_copy(tbl.at[idx_s.at[b + 2]],
                                         rows.at[jf], gsems.at[jf])

                    pltpu.make_async_copy(tbl.at[idx_s.at[b]], rows.at[j4],
                                          gsems.at[j4]).wait()
                    pltpu.async_copy(rows.at[j4], acc.at[idx_d.at[b]],
                                     ssems.at[j4], add=True)
                return carry

            lax.fori_loop(0, HB // NBUF, body, 0)
            for j in range(2):
                b = HB - 2 + j
                pltpu.make_async_copy(rows.at[b % NBUF],
                                      acc.at[idx_d.at[b]],
                                      ssems.at[b % NBUF]).wait()
        plsc.subcore_barrier()
        pltpu.sync_copy(acc.at[pl.ds(s * RPS, RPS)],
                        out.at[c, ch, pl.ds(s * RPS, RPS)])
        plsc.subcore_barrier()


def _sc_agg(y3, srcr, dstr):
    nchunk = y3.shape[1]
    return pl.kernel(
        functools.partial(_agg_body, nchunk),
        out_type=jax.ShapeDtypeStruct((NB, nchunk, N2, 128), _f32),
        mesh=_mesh,
        scratch_types=[
            pltpu.VMEM((HB, KB), jnp.int32),
            pltpu.VMEM((HB, KB), jnp.int32),
            pltpu.VMEM((NBUF, KB, 128), _f32),
            pltpu.VMEM_SHARED((N2, 128), _f32),
            pltpu.SemaphoreType.DMA((NBUF,)),
            pltpu.SemaphoreType.DMA((NBUF,)),
        ],
    )(y3, srcr, dstr)


def _segmax_body(z2, batch, neg_hbm, out, buf, bsm, acc, sems):
    c = lax.axis_index("c")
    s = lax.axis_index("s")
    pltpu.sync_copy(neg_hbm, acc)
    pltpu.sync_copy(batch.at[c, 0], bsm)
    cols = pl.ds(s * 128, 128)
    for j in range(2):
        pltpu.async_copy(z2.at[c, pl.ds(j * RSEG, RSEG), cols], buf.at[j],
                         sems.at[j])

    def _consume(r, j):
        def grp_body(gi, carry2):
            gvec = bsm[pl.ds(r * RSEG + gi * 16, 16)]
            for jj in range(16):
                g = gvec[jj]
                for v in range(8):
                    sl = pl.ds(v * 16, 16)
                    acc[g, sl] = jnp.maximum(acc[g, sl],
                                             buf[j, gi * 16 + jj, sl])
            return carry2

        lax.fori_loop(0, RSEG // 16, grp_body, 0)

    def chunk_body(t, carry):
        for j in range(2):
            r = t * 2 + j
            pltpu.make_async_copy(z2.at[c, pl.ds(r * RSEG, RSEG), cols],
                                  buf.at[j], sems.at[j]).wait()
            _consume(r, j)

            @pl.when(r + 2 < NSEG_CH)
            def _():
                pltpu.async_copy(z2.at[c, pl.ds((r + 2) * RSEG, RSEG), cols],
                                 buf.at[j], sems.at[j])
        return carry

    lax.fori_loop(0, NSEG_CH // 2, chunk_body, 0)
    rl = NSEG_CH - 1
    pltpu.make_async_copy(z2.at[c, pl.ds(rl * RSEG, RSEG), cols],
                          buf.at[rl % 2], sems.at[rl % 2]).wait()
    _consume(rl, rl % 2)
    pltpu.sync_copy(acc, out.at[c, :, pl.ds(s * 128, 128)])


def _sc_segmax(z2, batch, neg):
    return pl.kernel(
        _segmax_body,
        out_type=jax.ShapeDtypeStruct((NB, G, 2048), _f32),
        mesh=_mesh,
        scratch_types=[
            pltpu.VMEM((2, RSEG, 128), _f32),
            pltpu.VMEM((N,), jnp.int32),
            pltpu.VMEM((G, 128), _f32),
            pltpu.SemaphoreType.DMA((2,)),
        ],
    )(z2, batch, neg)


def _prescale_body(x_ref, deg_ref, dinv_ref, y1_ref):
    d = deg_ref[0, :, 0:1] + 1.0
    dv = lax.rsqrt(d)
    dinv_ref[0] = dv
    xv = x_ref[0] * dv
    for ci in range(4):
        y1_ref[0, ci] = xv[:, 128 * ci:128 * (ci + 1)]


def _tc_prescale(x2, deg2):
    nI = N // BI
    return pl.pallas_call(
        _prescale_body,
        grid=(NB, nI),
        in_specs=[
            pl.BlockSpec((1, BI, 512), lambda b, i: (b, i, 0)),
            pl.BlockSpec((1, BI, 128), lambda b, i: (b, i, 0)),
        ],
        out_specs=[
            pl.BlockSpec((1, BI, 1), lambda b, i: (b, i, 0)),
            pl.BlockSpec((1, 4, BI, 128), lambda b, i: (b, 0, i, 0)),
        ],
        out_shape=[
            jax.ShapeDtypeStruct((NB, N, 1), _f32),
            jax.ShapeDtypeStruct((NB, 4, N2, 128), _f32),
        ],
    )(x2, deg2)


def _mm_body(a_ref, w_ref, b_ref, dinv_ref, out_ref, *,
             cin, relu, post, chunked):
    acc = jnp.dot(a_ref[0, 0], w_ref[0][:128],
                  preferred_element_type=_f32)
    for ci in range(1, cin):
        acc += jnp.dot(a_ref[0, ci], w_ref[0][128 * ci:128 * (ci + 1)],
                       preferred_element_type=_f32)
    dv = dinv_ref[0]
    t = acc * dv + b_ref[0]
    if relu:
        t = jnp.maximum(t, 0.0)
    if post:
        t = t * dv
    if chunked:
        for q in range(BJ // 128):
            out_ref[0, q] = t[:, 128 * q:128 * (q + 1)]
    else:
        out_ref[0] = t


def _tc_gcn_matmul(a3, w2, b2, dinv2, relu, post, chunked):
    cin = a3.shape[1]
    wout = w2.shape[2]
    nI, nJ = N // BI, wout // BJ
    body = functools.partial(_mm_body, cin=cin, relu=relu, post=post,
                             chunked=chunked)
    nq = BJ // 128
    if chunked:
        out_spec = pl.BlockSpec((1, nq, BI, 128), lambda b, i, j: (b, j, i, 0))
        out_shape = jax.ShapeDtypeStruct((NB, wout // 128, N2, 128), _f32)
    else:
        out_spec = pl.BlockSpec((1, BI, BJ), lambda b, i, j: (b, i, j))
        out_shape = jax.ShapeDtypeStruct((NB, N, wout), _f32)
    return pl.pallas_call(
        body,
        grid=(NB, nI, nJ),
        in_specs=[
            pl.BlockSpec((1, cin, BI, 128), lambda b, i, j: (b, 0, i, 0)),
            pl.BlockSpec((1, 128 * cin, BJ), lambda b, i, j: (b, 0, j)),
            pl.BlockSpec((1, 1, BJ), lambda b, i, j: (b, 0, j)),
            pl.BlockSpec((1, BI, 1), lambda b, i, j: (b, i, 0)),
        ],
        out_specs=out_spec,
        out_shape=out_shape,
        compiler_params=pltpu.CompilerParams(
            dimension_semantics=("parallel", "parallel", "parallel")),
    )(a3, w2, b2, dinv2)


def _head_body(p_ref, w1_ref, b1_ref, w2_ref, b2_ref, o1_ref, o2_ref):
    q = p_ref[0] + p_ref[1]
    z = jnp.maximum(jnp.dot(q, w1_ref[...], preferred_element_type=_f32)
                    + b1_ref[...], 0.0)
    o = jnp.dot(z, w2_ref[...], preferred_element_type=_f32) + b2_ref[...]
    o1_ref[...] = o
    o2_ref[...] = jax.nn.sigmoid(o)


def _tc_head(p2, w1, b1, w2p, b2p):
    nout = w2p.shape[1]
    return pl.pallas_call(
        _head_body,
        out_shape=[
            jax.ShapeDtypeStruct((G, nout), _f32),
            jax.ShapeDtypeStruct((G, nout), _f32),
        ],
    )(p2, w1, b1, w2p, b2p)


def _prep_edges(ei):
    pad = EP - E
    src = jnp.concatenate([ei[0], jnp.zeros((pad,), jnp.int32)])
    dst = jnp.concatenate([ei[1], jnp.full((pad,), N + 8, jnp.int32)])
    return src.reshape(NS, NBATCH, KB), dst.reshape(NS, NBATCH, KB)


def kernel(x_s, x_t, edge_index_s, edge_index_t, xs_batch, xt_batch,
           W_enc1, b_enc1, W_enc2, b_enc2,
           W_r1a, b_r1a, W_r1b, b_r1b,
           W_r2a, b_r2a, W_r2b, b_r2b,
           W_l1, b_l1, W_l2, b_l2):
    x2 = jnp.stack([x_s, x_t])
    src_s, dst_s = _prep_edges(edge_index_s)
    src_t, dst_t = _prep_edges(edge_index_t)
    src2 = jnp.stack([src_s, src_t])
    dst2 = jnp.stack([dst_s, dst_t])
    batch2 = jnp.stack([xs_batch, xt_batch])[:, None, :]

    w_enc = jnp.stack([W_enc1, W_enc2])
    b_enc = jnp.stack([b_enc1, b_enc2])[:, None, :]
    w_ra = jnp.stack([W_r1a, W_r2a])
    b_ra = jnp.stack([b_r1a, b_r2a])[:, None, :]
    w_rb = jnp.stack([W_r1b, W_r2b])
    b_rb = jnp.stack([b_r1b, b_r2b])[:, None, :]

    b_l2r = b_l2[None, :]
    b_l1r = b_l1[None, :]

    zeros_init = jnp.zeros((N2, 128), _f32)
    ones_blk = jnp.ones((KB, 128), _f32)
    neg = jnp.full((G, 128), -jnp.inf, _f32)

    deg2 = _sc_degree(dst2, zeros_init, ones_blk)
    dinv2, y1 = _tc_prescale(x2, deg2)
    a1 = _sc_agg(y1, src2, dst2)
    y2 = _tc_gcn_matmul(a1, w_enc, b_enc, dinv2, relu=False, post=True,
                        chunked=True)
    a2 = _sc_agg(y2, src2, dst2)
    y3 = _tc_gcn_matmul(a2, w_ra, b_ra, dinv2, relu=True, post=True,
                        chunked=True)
    a3 = _sc_agg(y3, src2, dst2)
    z2 = _tc_gcn_matmul(a3, w_rb, b_rb, dinv2, relu=True, post=False,
                        chunked=False)
    p2 = _sc_segmax(z2, batch2, neg)
    z, sig = _tc_head(p2, W_l1, b_l1r, W_l2, b_l2r)
    return (z, sig)


# speedup vs baseline: 1.1695x; 1.0507x over previous
import functools

import jax
import jax.numpy as jnp
from jax import lax
from jax.experimental import pallas as pl
from jax.experimental.pallas import tpu as pltpu
from jax.experimental.pallas import tpu_sc as plsc

N = 10000
E = 160000
G = 64
NB = 2
NS = 16
KB = 64
NBATCH = 160
NBUF = 4
EP = NS * NBATCH * KB
N2 = 10112
RPS = N2 // NS
BI = 1000
BJ = 512
RSEG = 80
NSEG_CH = N // RSEG
NRES = 4
HB = NBATCH // NRES

_f32 = jnp.float32
_mesh = plsc.VectorSubcoreMesh(core_axis_name="c", subcore_axis_name="s")


def _deg_body(dstr, zeros_hbm, ones_hbm, out, idx_d, ones_v, acc):
    c = lax.axis_index("c")
    s = lax.axis_index("s")
    pltpu.sync_copy(dstr.at[c, s], idx_d)
    pltpu.sync_copy(ones_hbm, ones_v)
    pltpu.sync_copy(zeros_hbm.at[pl.ds(s * RPS, RPS)],
                    acc.at[pl.ds(s * RPS, RPS)])
    plsc.subcore_barrier()

    def body(b, carry):
        pltpu.sync_copy(ones_v, acc.at[idx_d.at[b]], add=True)
        return carry

    lax.fori_loop(0, NBATCH, body, 0)
    plsc.subcore_barrier()
    pltpu.sync_copy(acc.at[pl.ds(s * RPS, RPS)],
                    out.at[c, pl.ds(s * RPS, RPS)])


def _sc_degree(dstr, zeros_init, ones_blk):
    return pl.kernel(
        _deg_body,
        out_type=jax.ShapeDtypeStruct((NB, N2, 128), _f32),
        mesh=_mesh,
        scratch_types=[
            pltpu.VMEM((NBATCH, KB), jnp.int32),
            pltpu.VMEM((KB, 128), _f32),
            pltpu.VMEM_SHARED((N2, 128), _f32),
        ],
    )(dstr, zeros_init, ones_blk)


def _agg_body(nchunk, y3, srcr, dstr, out, idx_s, idx_d, rows, acc, gsems,
              ssems):
    c = lax.axis_index("c")
    s = lax.axis_index("s")
    half_chunks = nchunk // 2
    for cc in range(half_chunks):
        ch = c * half_chunks + cc
        pltpu.sync_copy(y3.at[ch, pl.ds(s * RPS, RPS)],
                        acc.at[pl.ds(s * RPS, RPS)])
        plsc.subcore_barrier()

        tbl = y3.at[ch]
        for half in range(NRES):
            pltpu.sync_copy(srcr.at[s, pl.ds(half * HB, HB)], idx_s)
            pltpu.sync_copy(dstr.at[s, pl.ds(half * HB, HB)], idx_d)
            for j in range(2):
                pltpu.async_copy(tbl.at[idx_s.at[j]], rows.at[j],
                                 gsems.at[j])

            def body(t, carry):
                for j4 in range(NBUF):
                    b = t * NBUF + j4
                    jf = (j4 + 2) % NBUF

                    @pl.when(b >= 2)
                    def _():
                        pltpu.make_async_copy(
                            rows.at[jf], acc.at[idx_d.at[b - 2]],
                            ssems.at[jf]).wait()

                    @pl.when(b + 2 < HB)
                    def _():
                        pltpu.async_copy(tbl.at[idx_s.at[b + 2]],
                                         rows.at[jf], gsems.at[jf])

                    pltpu.make_async_copy(tbl.at[idx_s.at[b]], rows.at[j4],
                                          gsems.at[j4]).wait()
                    pltpu.async_copy(rows.at[j4], acc.at[idx_d.at[b]],
                                     ssems.at[j4], add=True)
                return carry

            lax.fori_loop(0, HB // NBUF, body, 0)
            for j in range(2):
                b = HB - 2 + j
                pltpu.make_async_copy(rows.at[b % NBUF],
                                      acc.at[idx_d.at[b]],
                                      ssems.at[b % NBUF]).wait()
        plsc.subcore_barrier()
        pltpu.sync_copy(acc.at[pl.ds(s * RPS, RPS)],
                        out.at[ch, pl.ds(s * RPS, RPS)])
        plsc.subcore_barrier()


def _sc_agg(y3, srcr, dstr):
    nchunk = y3.shape[0]
    return pl.kernel(
        functools.partial(_agg_body, nchunk),
        out_type=jax.ShapeDtypeStruct((nchunk, N2, 128), _f32),
        mesh=_mesh,
        scratch_types=[
            pltpu.VMEM((HB, KB), jnp.int32),
            pltpu.VMEM((HB, KB), jnp.int32),
            pltpu.VMEM((NBUF, KB, 128), _f32),
            pltpu.VMEM_SHARED((N2, 128), _f32),
            pltpu.SemaphoreType.DMA((NBUF,)),
            pltpu.SemaphoreType.DMA((NBUF,)),
        ],
    )(y3, srcr, dstr)


def _segmax_body(z2s, z2t, batch, neg_hbm, out, buf, bsm, acc, sems):
    c = lax.axis_index("c")
    s = lax.axis_index("s")
    pltpu.sync_copy(neg_hbm, acc)
    pltpu.sync_copy(batch.at[c, 0], bsm)
    cols = pl.ds(s * 128, 128)

    def z2_issue(r, j):
        @pl.when(c == 0)
        def _():
            pltpu.async_copy(z2s.at[pl.ds(r * RSEG, RSEG), cols],
                             buf.at[j], sems.at[j])

        @pl.when(c == 1)
        def _():
            pltpu.async_copy(z2t.at[pl.ds(r * RSEG, RSEG), cols],
                             buf.at[j], sems.at[j])

    def z2_wait(r, j):
        @pl.when(c == 0)
        def _():
            pltpu.make_async_copy(z2s.at[pl.ds(r * RSEG, RSEG), cols],
                                  buf.at[j], sems.at[j]).wait()

        @pl.when(c == 1)
        def _():
            pltpu.make_async_copy(z2t.at[pl.ds(r * RSEG, RSEG), cols],
                                  buf.at[j], sems.at[j]).wait()

    for j in range(2):
        z2_issue(j, j)

    def _consume(r, j):
        def grp_body(gi, carry2):
            gvec = bsm[pl.ds(r * RSEG + gi * 16, 16)]
            for jj in range(16):
                g = gvec[jj]
                for v in range(8):
                    sl = pl.ds(v * 16, 16)
                    acc[g, sl] = jnp.maximum(acc[g, sl],
                                             buf[j, gi * 16 + jj, sl])
            return carry2

        lax.fori_loop(0, RSEG // 16, grp_body, 0)

    def chunk_body(t, carry):
        for j in range(2):
            r = t * 2 + j
            z2_wait(r, j)
            _consume(r, j)

            @pl.when(r + 2 < NSEG_CH)
            def _():
                z2_issue(r + 2, j)
        return carry

    lax.fori_loop(0, NSEG_CH // 2, chunk_body, 0)
    rl = NSEG_CH - 1
    z2_wait(rl, rl % 2)
    _consume(rl, rl % 2)
    pltpu.sync_copy(acc, out.at[c, :, pl.ds(s * 128, 128)])


def _sc_segmax(z2s, z2t, batch, neg):
    return pl.kernel(
        _segmax_body,
        out_type=jax.ShapeDtypeStruct((NB, G, 2048), _f32),
        mesh=_mesh,
        scratch_types=[
            pltpu.VMEM((2, RSEG, 128), _f32),
            pltpu.VMEM((N,), jnp.int32),
            pltpu.VMEM((G, 128), _f32),
            pltpu.SemaphoreType.DMA((2,)),
        ],
    )(z2s, z2t, batch, neg)


def _prescale_body(x_ref, deg_ref, dvs_ref, dvt_ref, ys_ref, yt_ref):
    for br, (dv_ref, y_ref) in enumerate(((dvs_ref, ys_ref),
                                          (dvt_ref, yt_ref))):
        d = deg_ref[br, :, 0:1] + 1.0
        dv = lax.rsqrt(d)
        dv_ref[...] = dv
        xv = x_ref[br] * dv
        for ci in range(4):
            y_ref[ci] = xv[:, 128 * ci:128 * (ci + 1)]


def _tc_prescale(x2, deg2):
    nI = N // BI
    return pl.pallas_call(
        _prescale_body,
        grid=(nI,),
        in_specs=[
            pl.BlockSpec((NB, BI, 512), lambda i: (0, i, 0)),
            pl.BlockSpec((NB, BI, 128), lambda i: (0, i, 0)),
        ],
        out_specs=[
            pl.BlockSpec((BI, 1), lambda i: (i, 0)),
            pl.BlockSpec((BI, 1), lambda i: (i, 0)),
            pl.BlockSpec((4, BI, 128), lambda i: (0, i, 0)),
            pl.BlockSpec((4, BI, 128), lambda i: (0, i, 0)),
        ],
        out_shape=[
            jax.ShapeDtypeStruct((N, 1), _f32),
            jax.ShapeDtypeStruct((N, 1), _f32),
            jax.ShapeDtypeStruct((4, N2, 128), _f32),
            jax.ShapeDtypeStruct((4, N2, 128), _f32),
        ],
    )(x2, deg2)


def _mm_body(a_ref, w_ref, b_ref, dinv_ref, out_ref, *,
             cin, relu, post, chunked):
    acc = jnp.dot(a_ref[0], w_ref[:128], preferred_element_type=_f32)
    for ci in range(1, cin):
        acc += jnp.dot(a_ref[ci], w_ref[128 * ci:128 * (ci + 1)],
                       preferred_element_type=_f32)
    dv = dinv_ref[...]
    t = acc * dv + b_ref[...]
    if relu:
        t = jnp.maximum(t, 0.0)
    if post:
        t = t * dv
    if chunked:
        for q in range(BJ // 128):
            out_ref[q] = t[:, 128 * q:128 * (q + 1)]
    else:
        out_ref[...] = t


def _tc_gcn_matmul(a3, w, bvec, dinv, relu, post, chunked):
    cin = a3.shape[0]
    wout = w.shape[1]
    nI, nJ = N // BI, wout // BJ
    body = functools.partial(_mm_body, cin=cin, relu=relu, post=post,
                             chunked=chunked)
    nq = BJ // 128
    if chunked:
        out_spec = pl.BlockSpec((nq, BI, 128), lambda i, j: (j, i, 0))
        out_shape = jax.ShapeDtypeStruct((wout // 128, N2, 128), _f32)
    else:
        out_spec = pl.BlockSpec((BI, BJ), lambda i, j: (i, j))
        out_shape = jax.ShapeDtypeStruct((N, wout), _f32)
    return pl.pallas_call(
        body,
        grid=(nI, nJ),
        in_specs=[
            pl.BlockSpec((cin, BI, 128), lambda i, j: (0, i, 0)),
            pl.BlockSpec((128 * cin, BJ), lambda i, j: (0, j)),
            pl.BlockSpec((1, BJ), lambda i, j: (0, j)),
            pl.BlockSpec((BI, 1), lambda i, j: (i, 0)),
        ],
        out_specs=out_spec,
        out_shape=out_shape,
        compiler_params=pltpu.CompilerParams(
            dimension_semantics=("parallel", "parallel")),
    )(a3, w, bvec, dinv)


def _head_body(p_ref, w1_ref, b1_ref, w2_ref, b2_ref, o1_ref, o2_ref):
    q = p_ref[0] + p_ref[1]
    z = jnp.maximum(jnp.dot(q, w1_ref[...], preferred_element_type=_f32)
                    + b1_ref[...], 0.0)
    o = jnp.dot(z, w2_ref[...], preferred_element_type=_f32) + b2_ref[...]
    o1_ref[...] = o
    o2_ref[...] = jax.nn.sigmoid(o)


def _tc_head(p2, w1, b1, w2, b2):
    nout = w2.shape[1]
    return pl.pallas_call(
        _head_body,
        out_shape=[
            jax.ShapeDtypeStruct((G, nout), _f32),
            jax.ShapeDtypeStruct((G, nout), _f32),
        ],
    )(p2, w1, b1, w2, b2)


def _prep_edges(ei):
    pad = EP - E
    src = jnp.concatenate([ei[0], jnp.zeros((pad,), jnp.int32)])
    dst = jnp.concatenate([ei[1], jnp.full((pad,), N + 8, jnp.int32)])
    return src.reshape(NS, NBATCH, KB), dst.reshape(NS, NBATCH, KB)


def kernel(x_s, x_t, edge_index_s, edge_index_t, xs_batch, xt_batch,
           W_enc1, b_enc1, W_enc2, b_enc2,
           W_r1a, b_r1a, W_r1b, b_r1b,
           W_r2a, b_r2a, W_r2b, b_r2b,
           W_l1, b_l1, W_l2, b_l2):
    x2 = jnp.stack([x_s, x_t])
    src_s, dst_s = _prep_edges(edge_index_s)
    src_t, dst_t = _prep_edges(edge_index_t)
    dst2 = jnp.stack([dst_s, dst_t])
    batch2 = jnp.stack([xs_batch, xt_batch])[:, None, :]

    zeros_init = jnp.zeros((N2, 128), _f32)
    ones_blk = jnp.ones((KB, 128), _f32)
    neg = jnp.full((G, 128), -jnp.inf, _f32)

    deg2 = _sc_degree(dst2, zeros_init, ones_blk)
    dinv_s, dinv_t, y1s, y1t = _tc_prescale(x2, deg2)

    a1s = _sc_agg(y1s, src_s, dst_s)
    a1t = _sc_agg(y1t, src_t, dst_t)
    y2s = _tc_gcn_matmul(a1s, W_enc1, b_enc1[None, :], dinv_s,
                         relu=False, post=True, chunked=True)
    y2t = _tc_gcn_matmul(a1t, W_enc2, b_enc2[None, :], dinv_t,
                         relu=False, post=True, chunked=True)

    a2s = _sc_agg(y2s, src_s, dst_s)
    a2t = _sc_agg(y2t, src_t, dst_t)
    y3s = _tc_gcn_matmul(a2s, W_r1a, b_r1a[None, :], dinv_s,
                         relu=True, post=True, chunked=True)
    y3t = _tc_gcn_matmul(a2t, W_r2a, b_r2a[None, :], dinv_t,
                         relu=True, post=True, chunked=True)

    a3s = _sc_agg(y3s, src_s, dst_s)
    a3t = _sc_agg(y3t, src_t, dst_t)
    z2s = _tc_gcn_matmul(a3s, W_r1b, b_r1b[None, :], dinv_s,
                         relu=True, post=False, chunked=False)
    z2t = _tc_gcn_matmul(a3t, W_r2b, b_r2b[None, :], dinv_t,
                         relu=True, post=False, chunked=False)

    p2 = _sc_segmax(z2s, z2t, batch2, neg)
    z, sig = _tc_head(p2, W_l1, b_l1[None, :], W_l2, b_l2[None, :])
    return (z, sig)
